# baseline (speedup 1.0000x reference)
import functools

import jax
import jax.numpy as jnp
from jax import lax
from jax.experimental import pallas as pl
from jax.experimental.pallas import tpu as pltpu
from jax.experimental.pallas import tpu_sc as plsc

T, N, D = 4, 10000, 128
E = 320000
D_EDGE = 16
POS = 3
HID = 64
NTOT = T * N

F32 = jnp.float32


def _silu(v):
    return v / (1.0 + jnp.exp(-v))


def _dot(a, b):
    return jnp.dot(a, b, preferred_element_type=F32)


def _spectral_x_body(x_ref, w0r_ref, w1r_ref, w1i_ref, wd_ref, wq_ref,
                     x2_ref, p_ref, q_ref):
    x0 = x_ref[0]
    x1 = x_ref[1]
    x2 = x_ref[2]
    x3 = x_ref[3]
    f0 = x0 + x1 + x2 + x3
    a = x0 - x2
    b = x3 - x1
    r0 = _dot(f0, w0r_ref[...])
    r1 = _dot(a, w1r_ref[...]) - _dot(b, w1i_ref[...])
    i1 = _dot(a, w1i_ref[...]) + _dot(b, w1r_ref[...])
    y0 = 0.25 * (r0 + 2.0 * r1)
    y1 = 0.25 * (r0 - 2.0 * i1)
    y2 = 0.25 * (r0 - 2.0 * r1)
    y3 = 0.25 * (r0 + 2.0 * i1)
    o0 = x0 + y0
    o1 = x1 + y1
    o2 = x2 + y2
    o3 = x3 + y3
    x2_ref[0] = o0
    x2_ref[1] = o1
    x2_ref[2] = o2
    x2_ref[3] = o3
    wd = wd_ref[...]
    wq = wq_ref[...]
    p_ref[0] = _dot(o0, wd)
    p_ref[1] = _dot(o1, wd)
    p_ref[2] = _dot(o2, wd)
    p_ref[3] = _dot(o3, wd)
    q_ref[0] = _dot(o0, wq)
    q_ref[1] = _dot(o1, wq)
    q_ref[2] = _dot(o2, wq)
    q_ref[3] = _dot(o3, wq)


def _spectral_x(x, w0r, w1r, w1i, wd, wq):
    nb = 1000
    grid = N // nb
    full = lambda shape: pl.BlockSpec(shape, lambda i: (0,) * len(shape))
    return pl.pallas_call(
        _spectral_x_body,
        grid=(grid,),
        in_specs=[
            pl.BlockSpec((T, nb, D), lambda i: (0, i, 0)),
            full((D, D)), full((D, D)), full((D, D)),
            full((D, HID)), full((D, HID)),
        ],
        out_specs=[
            pl.BlockSpec((T, nb, D), lambda i: (0, i, 0)),
            pl.BlockSpec((T, nb, HID), lambda i: (0, i, 0)),
            pl.BlockSpec((T, nb, HID), lambda i: (0, i, 0)),
        ],
        out_shape=[
            jax.ShapeDtypeStruct((T, N, D), F32),
            jax.ShapeDtypeStruct((T, N, HID), F32),
            jax.ShapeDtypeStruct((T, N, HID), F32),
        ],
    )(x, w0r, w1r, w1i, wd, wq)


def _center_body(pos_ref, out_ref):
    i = pl.program_id(0)

    @pl.when(i == 0)
    def _init():
        out_ref[...] = jnp.zeros_like(out_ref)

    part = jnp.sum(pos_ref[...], axis=1)
    out_ref[...] += part * (1.0 / N)


def _center(pos):
    nb = 1000
    return pl.pallas_call(
        _center_body,
        grid=(N // nb,),
        in_specs=[pl.BlockSpec((T, nb, POS), lambda i: (0, i, 0))],
        out_specs=pl.BlockSpec((T, POS), lambda i: (0, 0)),
        out_shape=jax.ShapeDtypeStruct((T, POS), F32),
    )(pos)


def _spectral_v_body(pos_ref, vel_ref, c_ref, wr_ref, wi_ref, pf_ref, vf_ref):
    center = []
    pc = []
    vv = []
    for t in range(T):
        ct = c_ref[t]
        center.append(ct)
        pc.append(pos_ref[t] - ct)
        vv.append(vel_ref[t])
    f0p = pc[0] + pc[1] + pc[2] + pc[3]
    f0v = vv[0] + vv[1] + vv[2] + vv[3]
    ap = pc[0] - pc[2]
    av = vv[0] - vv[2]
    bp = pc[3] - pc[1]
    bv = vv[3] - vv[1]

    def mix(arr_p, arr_v, w_ref, o, m):
        return arr_p * w_ref[4 * 0 + 2 * o + m] + arr_v * w_ref[4 * 1 + 2 * o + m]

    for o in range(2):
        r0 = mix(f0p, f0v, wr_ref, o, 0)
        r1 = mix(ap, av, wr_ref, o, 1) - mix(bp, bv, wi_ref, o, 1)
        i1 = mix(ap, av, wi_ref, o, 1) + mix(bp, bv, wr_ref, o, 1)
        y = (0.25 * (r0 + 2.0 * r1), 0.25 * (r0 - 2.0 * i1),
             0.25 * (r0 - 2.0 * r1), 0.25 * (r0 + 2.0 * i1))
        for t in range(T):
            if o == 0:
                pf_ref[t, :, 0:POS] = pc[t] + y[t] + center[t]
            else:
                vf_ref[t, :, 0:POS] = vv[t] + y[t]
    nb = pf_ref.shape[1]
    pf_ref[:, :, POS:] = jnp.zeros((T, nb, 16 - POS), F32)
    vf_ref[:, :, POS:] = jnp.zeros((T, nb, 16 - POS), F32)


def _spectral_v(pos, vel, center, wr8, wi8):
    nb = 1000
    return pl.pallas_call(
        _spectral_v_body,
        grid=(N // nb,),
        in_specs=[
            pl.BlockSpec((T, nb, POS), lambda i: (0, i, 0)),
            pl.BlockSpec((T, nb, POS), lambda i: (0, i, 0)),
            pl.BlockSpec((T, POS), lambda i: (0, 0)),
            pl.BlockSpec(memory_space=pltpu.SMEM),
            pl.BlockSpec(memory_space=pltpu.SMEM),
        ],
        out_specs=[
            pl.BlockSpec((T, nb, 16), lambda i: (0, i, 0)),
            pl.BlockSpec((T, nb, 16), lambda i: (0, i, 0)),
        ],
        out_shape=[
            jax.ShapeDtypeStruct((T, N, 16), F32),
            jax.ShapeDtypeStruct((T, N, 16), F32),
        ],
    )(pos, vel, center, wr8, wi8)


def _edge_body(hp_ref, hq_ref, dpd_ref, dps_ref, ea_ref, wdist_ref, wea_ref,
               b1_ref, w2_ref, b2_ref, w3_ref, b3_ref,
               v1_ref, c1_ref, v2_ref, c2_ref, v3_ref, c3_ref,
               mpa_ref, mpb_ref):
    diff = dpd_ref[...] - dps_ref[...]
    dist = jnp.sqrt(jnp.sum(diff * diff, axis=1, keepdims=True) + 1e-12)
    u = hp_ref[...] + hq_ref[...] + _dot(ea_ref[...], wea_ref[...]) \
        + dist * wdist_ref[...] + b1_ref[...]
    u = _silu(u)
    u = _silu(_dot(u, w2_ref[...]) + b2_ref[...])
    m = _dot(u, w3_ref[...]) + b3_ref[...]
    p = _silu(_dot(m, v1_ref[...]) + c1_ref[...])
    p = _silu(_dot(p, v2_ref[...]) + c2_ref[...])
    s = _dot(p, v3_ref[...]) + c3_ref[...]
    mpa_ref[...] = m[:, 0:40]
    mpb_ref[:, 0:24] = m[:, 40:64]
    mpb_ref[:, 24:40] = diff * s


def _edge_mlp(hp, hq, dpd, dps, ea, wdist, wea, b1, w2, b2, w3, b3,
              v1, c1, v2, c2, v3, c3):
    eb = 3200
    grid = E // eb
    full = lambda shape: pl.BlockSpec(shape, lambda i: (0,) * len(shape))
    return pl.pallas_call(
        _edge_body,
        grid=(grid,),
        in_specs=[
            pl.BlockSpec((eb, HID), lambda i: (i, 0)),
            pl.BlockSpec((eb, HID), lambda i: (i, 0)),
            pl.BlockSpec((eb, 16), lambda i: (i, 0)),
            pl.BlockSpec((eb, 16), lambda i: (i, 0)),
            pl.BlockSpec((eb, D_EDGE), lambda i: (i, 0)),
            full((1, HID)), full((D_EDGE, HID)), full((1, HID)),
            full((HID, HID)), full((1, HID)),
            full((HID, HID)), full((1, HID)),
            full((HID, HID)), full((1, HID)),
            full((HID, HID)), full((1, HID)),
            full((HID, 1)), full((1, 1)),
        ],
        out_specs=[
            pl.BlockSpec((eb, _AGW), lambda i: (i, 0)),
            pl.BlockSpec((eb, _AGW), lambda i: (i, 0)),
        ],
        out_shape=[
            jax.ShapeDtypeStruct((E, _AGW), F32),
            jax.ShapeDtypeStruct((E, _AGW), F32),
        ],
    )(hp, hq, dpd, dps, ea, wdist, wea, b1, w2, b2, w3, b3,
      v1, c1, v2, c2, v3, c3)


def _node_body(xf_ref, aga_ref, agb_ref, pf_ref, vf_ref,
               u1a_ref, u1ba_ref, u1bb_ref, fb1_ref, u2_ref, fb2_ref,
               u3_ref, fb3_ref,
               z1_ref, zb1_ref, z2_ref, zb2_ref, z3_ref, zb3_ref,
               xn_ref, pn_ref, vn_ref):
    xf = xf_ref[...]
    aga = aga_ref[...]
    agb = agb_ref[...]
    ap = agb[:, 24:40]
    h = _silu(_dot(xf, u1a_ref[...]) + _dot(aga, u1ba_ref[...])
              + _dot(agb[:, 0:24], u1bb_ref[...]) + fb1_ref[...])
    h = _silu(_dot(h, u2_ref[...]) + fb2_ref[...])
    xn_ref[...] = _dot(h, u3_ref[...]) + fb3_ref[...]
    z = _silu(_dot(xf, z1_ref[...]) + zb1_ref[...])
    z = _silu(_dot(z, z2_ref[...]) + zb2_ref[...])
    s = _dot(z, z3_ref[...]) + zb3_ref[...]
    vn = s * vf_ref[...] + ap
    vn_ref[...] = vn
    pn_ref[...] = pf_ref[...] + vn


def _node_update(xf, aga, agb, pf16, vf16, u1a, u1ba, u1bb, fb1, u2, fb2,
                 u3, fb3, z1, zb1, z2, zb2, z3, zb3):
    nb = 2000
    grid = NTOT // nb
    full = lambda shape: pl.BlockSpec(shape, lambda i: (0,) * len(shape))
    return pl.pallas_call(
        _node_body,
        grid=(grid,),
        in_specs=[
            pl.BlockSpec((nb, D), lambda i: (i, 0)),
            pl.BlockSpec((nb, _AGW), lambda i: (i, 0)),
            pl.BlockSpec((nb, _AGW), lambda i: (i, 0)),
            pl.BlockSpec((nb, 16), lambda i: (i, 0)),
            pl.BlockSpec((nb, 16), lambda i: (i, 0)),
            full((D, HID)), full((_AGW, HID)), full((24, HID)),
            full((1, HID)),
            full((HID, HID)), full((1, HID)),
            full((HID, D)), full((1, D)),
            full((D, HID)), full((1, HID)),
            full((HID, HID)), full((1, HID)),
            full((HID, 1)), full((1, 1)),
        ],
        out_specs=[
            pl.BlockSpec((nb, D), lambda i: (i, 0)),
            pl.BlockSpec((nb, 16), lambda i: (i, 0)),
            pl.BlockSpec((nb, 16), lambda i: (i, 0)),
        ],
        out_shape=[
            jax.ShapeDtypeStruct((NTOT, D), F32),
            jax.ShapeDtypeStruct((NTOT, 16), F32),
            jax.ShapeDtypeStruct((NTOT, 16), F32),
        ],
    )(xf, aga, agb, pf16, vf16, u1a, u1ba, u1bb, fb1, u2, fb2, u3, fb3,
      z1, zb1, z2, zb2, z3, zb3)


_NC = 2
_NS = 16
_NW = _NC * _NS
_GC = 80
_EPW = E // _NW
_GNCH = _EPW // _GC


_EPT = E // _NS
_GCH = 400
_GSUB = 80


def _gather_body(p_hbm, q_hbm, pf_hbm, src_hbm, dst_hbm,
                 hp_out, hq_out, fd_out, fs_out,
                 idxbuf, hbuf, fbuf, s0, s1):
    cid = lax.axis_index("c")
    sid = lax.axis_index("s")

    def pump(idx_hbm, tab_hbm, out_h, out_f):
        pltpu.sync_copy(idx_hbm.at[pl.ds(sid * _EPT, _EPT)], idxbuf)

        def chunk(j, carry):
            off = j * _GCH
            cps = []
            for k in range(_GCH // _GSUB):
                sl = pl.ds(off + k * _GSUB, _GSUB)
                dsl = pl.ds(k * _GSUB, _GSUB)
                cps.append(pltpu.async_copy(
                    tab_hbm.at[idxbuf.at[sl]], hbuf.at[dsl], s0))
                cps.append(pltpu.async_copy(
                    pf_hbm.at[idxbuf.at[sl]], fbuf.at[dsl], s1))
            for cp in cps:
                cp.wait()
            base = sid * _EPT + off
            pltpu.sync_copy(hbuf, out_h.at[pl.ds(base, _GCH)])
            pltpu.sync_copy(fbuf, out_f.at[pl.ds(base, _GCH)])
            return carry

        lax.fori_loop(0, _EPT // _GCH, chunk, 0)

    @pl.when(cid == 0)
    def _core0():
        pump(dst_hbm, p_hbm, hp_out, fd_out)

    @pl.when(cid == 1)
    def _core1():
        pump(src_hbm, q_hbm, hq_out, fs_out)


def _gather_stage(p2, q2, pf16f, src, dst):
    mesh = plsc.VectorSubcoreMesh(core_axis_name="c", subcore_axis_name="s")
    k = pl.kernel(
        _gather_body, mesh=mesh,
        compiler_params=pltpu.CompilerParams(use_tc_tiling_on_sc=False),
        out_type=[
            jax.ShapeDtypeStruct((E, HID), F32),
            jax.ShapeDtypeStruct((E, HID), F32),
            jax.ShapeDtypeStruct((E, 16), F32),
            jax.ShapeDtypeStruct((E, 16), F32),
        ],
        scratch_types=[
            pltpu.VMEM((_EPT,), jnp.int32),
            pltpu.VMEM((_GCH, HID), F32),
            pltpu.VMEM((_GCH, 16), F32),
            pltpu.SemaphoreType.DMA,
            pltpu.SemaphoreType.DMA,
        ],
    )
    return k(p2, q2, pf16f, src, dst)


_SC_ROWS = 40960
_SC_RPT = _SC_ROWS // _NS
_SCH = 80
_SNCH = _EPT // _SCH
_AGW = 40


def _scatter_body(mpa_hbm, mpb_hbm, dst_hbm, aga_out, agb_out,
                  didx, mpbuf, zbuf, shared, s0):
    cid = lax.axis_index("c")
    sid = lax.axis_index("s")

    def zrow(r, c):
        z = jnp.zeros((16,), F32)
        zbuf[r, pl.ds(0, 16)] = z
        zbuf[r, pl.ds(16, 16)] = z
        zbuf[r, pl.ds(24, 16)] = z
        return c

    lax.fori_loop(0, 320, zrow, 0)
    for k in range(_SC_RPT // 320):
        pltpu.sync_copy(zbuf, shared.at[pl.ds(sid * _SC_RPT + k * 320, 320)])
    plsc.subcore_barrier()

    def pump(mp_hbm):
        def chunk(j, carry):
            base = sid * _EPT + j * _SCH
            pltpu.sync_copy(dst_hbm.at[pl.ds(base, _SCH)], didx)
            cp = pltpu.async_copy(mp_hbm.at[pl.ds(base, _SCH)], mpbuf, s0)
            cp.wait()
            pltpu.sync_copy(mpbuf, shared.at[didx], add=True)
            return carry

        lax.fori_loop(0, _SNCH, chunk, 0)

    @pl.when(cid == 0)
    def _core0():
        pump(mpa_hbm)

    @pl.when(cid == 1)
    def _core1():
        pump(mpb_hbm)

    plsc.subcore_barrier()

    for k in range(_SC_RPT // 320):
        row = sid * _SC_RPT + k * 320

        @pl.when(row < NTOT)
        def _cp():
            @pl.when(cid == 0)
            def _a():
                pltpu.sync_copy(shared.at[pl.ds(row, 320)],
                                aga_out.at[pl.ds(row, 320)])

            @pl.when(cid == 1)
            def _b():
                pltpu.sync_copy(shared.at[pl.ds(row, 320)],
                                agb_out.at[pl.ds(row, 320)])


def _scatter_stage(mpa, mpb, dst):
    mesh = plsc.VectorSubcoreMesh(core_axis_name="c", subcore_axis_name="s")
    k = pl.kernel(
        _scatter_body, mesh=mesh,
        compiler_params=pltpu.CompilerParams(use_tc_tiling_on_sc=False),
        out_type=[
            jax.ShapeDtypeStruct((NTOT, _AGW), F32),
            jax.ShapeDtypeStruct((NTOT, _AGW), F32),
        ],
        scratch_types=[
            pltpu.VMEM((_SCH,), jnp.int32),
            pltpu.VMEM((_SCH, _AGW), F32),
            pltpu.VMEM((320, _AGW), F32),
            pltpu.VMEM_SHARED((_SC_ROWS, _AGW), F32),
            pltpu.SemaphoreType.DMA,
        ],
    )
    return k(mpa, mpb, dst)


def kernel(x, pos, vel, edge_index, edge_attr, params):
    w0r = params['weight_scalar_r'][:, :, 0]
    w1r = params['weight_scalar_r'][:, :, 1]
    w1i = params['weight_scalar_i'][:, :, 1]
    wvr8 = params['weight_vector_r'].reshape(8)
    wvi8 = params['weight_vector_i'].reshape(8)

    (mw1, mb1), (mw2, mb2), (mw3, mb3) = params['message_net']
    wd = mw1[0:D]
    wq = mw1[D:2 * D]
    wdist = mw1[2 * D:2 * D + 1]
    wea = mw1[2 * D + 1:]

    (pv1, pc1), (pv2, pc2), (pv3, pc3) = params['update_pos_net']
    (fu1, fb1), (fu2, fb2), (fu3, fb3) = params['update_feat_net']
    u1a = fu1[0:D]
    u1b = fu1[D:]
    (zv1, zb1), (zv2, zb2), (zv3, zb3) = params['update_vel_net']

    row = lambda b: b.reshape(1, -1)

    x2, p, q = _spectral_x(x, w0r, w1r, w1i, wd, wq)
    center = _center(pos)
    pf16, vf16 = _spectral_v(pos, vel, center, wvr8, wvi8)

    xf = x2.reshape(NTOT, D)
    p2 = p.reshape(NTOT, HID)
    q2 = q.reshape(NTOT, HID)
    pf16f = pf16.reshape(NTOT, 16)
    vf16f = vf16.reshape(NTOT, 16)
    src = edge_index[0]
    dst = edge_index[1]
    ea = edge_attr.reshape(E, D_EDGE)

    hp, hq, dpd, dps = _gather_stage(p2, q2, pf16f, src, dst)

    mpa, mpb = _edge_mlp(hp, hq, dpd, dps, ea, row(wdist.reshape(-1)), wea,
                         row(mb1), mw2, row(mb2), mw3, row(mb3),
                         pv1, row(pc1), pv2, row(pc2), pv3, row(pc3))

    aga, agb = _scatter_stage(mpa, mpb, dst)

    xn, pn16, vn16 = _node_update(
        xf, aga, agb, pf16f, vf16f,
        u1a, u1b[0:40], u1b[40:64], row(fb1), fu2, row(fb2), fu3, row(fb3),
        zv1, row(zb1), zv2, row(zb2), zv3, row(zb3))

    x_new = xn.reshape(T, N, D)
    pos_new = pn16[:, 0:POS].reshape(T, N, POS)
    vel_new = vn16[:, 0:POS].reshape(T, N, POS)
    return (x_new, pos_new, vel_new)


# speedup vs baseline: 1.1241x; 1.1241x over previous
import functools

import jax
import jax.numpy as jnp
from jax import lax
from jax.experimental import pallas as pl
from jax.experimental.pallas import tpu as pltpu
from jax.experimental.pallas import tpu_sc as plsc

T, N, D = 4, 10000, 128
E = 320000
D_EDGE = 16
POS = 3
HID = 64
NTOT = T * N

F32 = jnp.float32


def _silu(v):
    return v / (1.0 + jnp.exp(-v))


def _dot(a, b):
    return jnp.dot(a, b, preferred_element_type=F32)


def _spectral_x_body(x_ref, w0r_ref, w1r_ref, w1i_ref, wd_ref, wq_ref,
                     x2_ref, p_ref, q_ref):
    x0 = x_ref[0]
    x1 = x_ref[1]
    x2 = x_ref[2]
    x3 = x_ref[3]
    f0 = x0 + x1 + x2 + x3
    a = x0 - x2
    b = x3 - x1
    r0 = _dot(f0, w0r_ref[...])
    r1 = _dot(a, w1r_ref[...]) - _dot(b, w1i_ref[...])
    i1 = _dot(a, w1i_ref[...]) + _dot(b, w1r_ref[...])
    y0 = 0.25 * (r0 + 2.0 * r1)
    y1 = 0.25 * (r0 - 2.0 * i1)
    y2 = 0.25 * (r0 - 2.0 * r1)
    y3 = 0.25 * (r0 + 2.0 * i1)
    o0 = x0 + y0
    o1 = x1 + y1
    o2 = x2 + y2
    o3 = x3 + y3
    x2_ref[0] = o0
    x2_ref[1] = o1
    x2_ref[2] = o2
    x2_ref[3] = o3
    wd = wd_ref[...]
    wq = wq_ref[...]
    p_ref[0] = _dot(o0, wd)
    p_ref[1] = _dot(o1, wd)
    p_ref[2] = _dot(o2, wd)
    p_ref[3] = _dot(o3, wd)
    q_ref[0] = _dot(o0, wq)
    q_ref[1] = _dot(o1, wq)
    q_ref[2] = _dot(o2, wq)
    q_ref[3] = _dot(o3, wq)


def _spectral_x(x, w0r, w1r, w1i, wd, wq):
    nb = 1000
    grid = N // nb
    full = lambda shape: pl.BlockSpec(shape, lambda i: (0,) * len(shape))
    return pl.pallas_call(
        _spectral_x_body,
        grid=(grid,),
        in_specs=[
            pl.BlockSpec((T, nb, D), lambda i: (0, i, 0)),
            full((D, D)), full((D, D)), full((D, D)),
            full((D, HID)), full((D, HID)),
        ],
        out_specs=[
            pl.BlockSpec((T, nb, D), lambda i: (0, i, 0)),
            pl.BlockSpec((T, nb, HID), lambda i: (0, i, 0)),
            pl.BlockSpec((T, nb, HID), lambda i: (0, i, 0)),
        ],
        out_shape=[
            jax.ShapeDtypeStruct((T, N, D), F32),
            jax.ShapeDtypeStruct((T, N, HID), F32),
            jax.ShapeDtypeStruct((T, N, HID), F32),
        ],
    )(x, w0r, w1r, w1i, wd, wq)


def _center_body(pos_ref, out_ref):
    i = pl.program_id(0)

    @pl.when(i == 0)
    def _init():
        out_ref[...] = jnp.zeros_like(out_ref)

    part = jnp.sum(pos_ref[...], axis=1)
    out_ref[...] += part * (1.0 / N)


def _center(pos):
    nb = 1000
    return pl.pallas_call(
        _center_body,
        grid=(N // nb,),
        in_specs=[pl.BlockSpec((T, nb, POS), lambda i: (0, i, 0))],
        out_specs=pl.BlockSpec((T, POS), lambda i: (0, 0)),
        out_shape=jax.ShapeDtypeStruct((T, POS), F32),
    )(pos)


def _spectral_v_body(pos_ref, vel_ref, c_ref, wr_ref, wi_ref, pf_ref, vf_ref):
    center = []
    pc = []
    vv = []
    for t in range(T):
        ct = c_ref[t]
        center.append(ct)
        pc.append(pos_ref[t] - ct)
        vv.append(vel_ref[t])
    f0p = pc[0] + pc[1] + pc[2] + pc[3]
    f0v = vv[0] + vv[1] + vv[2] + vv[3]
    ap = pc[0] - pc[2]
    av = vv[0] - vv[2]
    bp = pc[3] - pc[1]
    bv = vv[3] - vv[1]

    def mix(arr_p, arr_v, w_ref, o, m):
        return arr_p * w_ref[4 * 0 + 2 * o + m] + arr_v * w_ref[4 * 1 + 2 * o + m]

    for o in range(2):
        r0 = mix(f0p, f0v, wr_ref, o, 0)
        r1 = mix(ap, av, wr_ref, o, 1) - mix(bp, bv, wi_ref, o, 1)
        i1 = mix(ap, av, wi_ref, o, 1) + mix(bp, bv, wr_ref, o, 1)
        y = (0.25 * (r0 + 2.0 * r1), 0.25 * (r0 - 2.0 * i1),
             0.25 * (r0 - 2.0 * r1), 0.25 * (r0 + 2.0 * i1))
        for t in range(T):
            if o == 0:
                pf_ref[t, :, 0:POS] = pc[t] + y[t] + center[t]
            else:
                vf_ref[t, :, 0:POS] = vv[t] + y[t]
    nb = pf_ref.shape[1]
    pf_ref[:, :, POS:] = jnp.zeros((T, nb, 16 - POS), F32)
    vf_ref[:, :, POS:] = jnp.zeros((T, nb, 16 - POS), F32)


def _spectral_v(pos, vel, center, wr8, wi8):
    nb = 1000
    return pl.pallas_call(
        _spectral_v_body,
        grid=(N // nb,),
        in_specs=[
            pl.BlockSpec((T, nb, POS), lambda i: (0, i, 0)),
            pl.BlockSpec((T, nb, POS), lambda i: (0, i, 0)),
            pl.BlockSpec((T, POS), lambda i: (0, 0)),
            pl.BlockSpec(memory_space=pltpu.SMEM),
            pl.BlockSpec(memory_space=pltpu.SMEM),
        ],
        out_specs=[
            pl.BlockSpec((T, nb, 16), lambda i: (0, i, 0)),
            pl.BlockSpec((T, nb, 16), lambda i: (0, i, 0)),
        ],
        out_shape=[
            jax.ShapeDtypeStruct((T, N, 16), F32),
            jax.ShapeDtypeStruct((T, N, 16), F32),
        ],
    )(pos, vel, center, wr8, wi8)


def _edge_body(hp_ref, hq_ref, dpd_ref, dps_ref, ea_ref, wdist_ref, wea_ref,
               b1_ref, w2_ref, b2_ref, w3_ref, b3_ref,
               v1_ref, c1_ref, v2_ref, c2_ref, v3_ref, c3_ref,
               mpa_ref, mpb_ref):
    diff = dpd_ref[...] - dps_ref[...]
    dist = jnp.sqrt(jnp.sum(diff * diff, axis=1, keepdims=True) + 1e-12)
    u = hp_ref[...] + hq_ref[...] + _dot(ea_ref[...], wea_ref[...]) \
        + dist * wdist_ref[...] + b1_ref[...]
    u = _silu(u)
    u = _silu(_dot(u, w2_ref[...]) + b2_ref[...])
    m = _dot(u, w3_ref[...]) + b3_ref[...]
    p = _silu(_dot(m, v1_ref[...]) + c1_ref[...])
    p = _silu(_dot(p, v2_ref[...]) + c2_ref[...])
    s = _dot(p, v3_ref[...]) + c3_ref[...]
    mpa_ref[...] = m[:, 0:40]
    mpb_ref[:, 0:24] = m[:, 40:64]
    mpb_ref[:, 24:40] = diff * s


def _edge_mlp(hp, hq, dpd, dps, ea, wdist, wea, b1, w2, b2, w3, b3,
              v1, c1, v2, c2, v3, c3):
    eb = 3200
    grid = E // eb
    full = lambda shape: pl.BlockSpec(shape, lambda i: (0,) * len(shape))
    return pl.pallas_call(
        _edge_body,
        grid=(grid,),
        in_specs=[
            pl.BlockSpec((eb, HID), lambda i: (i, 0)),
            pl.BlockSpec((eb, HID), lambda i: (i, 0)),
            pl.BlockSpec((eb, 16), lambda i: (i, 0)),
            pl.BlockSpec((eb, 16), lambda i: (i, 0)),
            pl.BlockSpec((eb, D_EDGE), lambda i: (i, 0)),
            full((1, HID)), full((D_EDGE, HID)), full((1, HID)),
            full((HID, HID)), full((1, HID)),
            full((HID, HID)), full((1, HID)),
            full((HID, HID)), full((1, HID)),
            full((HID, HID)), full((1, HID)),
            full((HID, 1)), full((1, 1)),
        ],
        out_specs=[
            pl.BlockSpec((eb, _AGW), lambda i: (i, 0)),
            pl.BlockSpec((eb, _AGW), lambda i: (i, 0)),
        ],
        out_shape=[
            jax.ShapeDtypeStruct((E, _AGW), F32),
            jax.ShapeDtypeStruct((E, _AGW), F32),
        ],
    )(hp, hq, dpd, dps, ea, wdist, wea, b1, w2, b2, w3, b3,
      v1, c1, v2, c2, v3, c3)


def _node_body(xf_ref, aga_ref, agb_ref, pf_ref, vf_ref,
               u1a_ref, u1ba_ref, u1bb_ref, fb1_ref, u2_ref, fb2_ref,
               u3_ref, fb3_ref,
               z1_ref, zb1_ref, z2_ref, zb2_ref, z3_ref, zb3_ref,
               xn_ref, pn_ref, vn_ref):
    xf = xf_ref[...]
    aga = aga_ref[...]
    agb = agb_ref[...]
    ap = agb[:, 24:40]
    h = _silu(_dot(xf, u1a_ref[...]) + _dot(aga, u1ba_ref[...])
              + _dot(agb[:, 0:24], u1bb_ref[...]) + fb1_ref[...])
    h = _silu(_dot(h, u2_ref[...]) + fb2_ref[...])
    xn_ref[...] = _dot(h, u3_ref[...]) + fb3_ref[...]
    z = _silu(_dot(xf, z1_ref[...]) + zb1_ref[...])
    z = _silu(_dot(z, z2_ref[...]) + zb2_ref[...])
    s = _dot(z, z3_ref[...]) + zb3_ref[...]
    vn = s * vf_ref[...] + ap
    vn_ref[...] = vn
    pn_ref[...] = pf_ref[...] + vn


def _node_update(xf, aga, agb, pf16, vf16, u1a, u1ba, u1bb, fb1, u2, fb2,
                 u3, fb3, z1, zb1, z2, zb2, z3, zb3):
    nb = 2000
    grid = NTOT // nb
    full = lambda shape: pl.BlockSpec(shape, lambda i: (0,) * len(shape))
    return pl.pallas_call(
        _node_body,
        grid=(grid,),
        in_specs=[
            pl.BlockSpec((nb, D), lambda i: (i, 0)),
            pl.BlockSpec((nb, _AGW), lambda i: (i, 0)),
            pl.BlockSpec((nb, _AGW), lambda i: (i, 0)),
            pl.BlockSpec((nb, 16), lambda i: (i, 0)),
            pl.BlockSpec((nb, 16), lambda i: (i, 0)),
            full((D, HID)), full((_AGW, HID)), full((24, HID)),
            full((1, HID)),
            full((HID, HID)), full((1, HID)),
            full((HID, D)), full((1, D)),
            full((D, HID)), full((1, HID)),
            full((HID, HID)), full((1, HID)),
            full((HID, 1)), full((1, 1)),
        ],
        out_specs=[
            pl.BlockSpec((nb, D), lambda i: (i, 0)),
            pl.BlockSpec((nb, 16), lambda i: (i, 0)),
            pl.BlockSpec((nb, 16), lambda i: (i, 0)),
        ],
        out_shape=[
            jax.ShapeDtypeStruct((NTOT, D), F32),
            jax.ShapeDtypeStruct((NTOT, 16), F32),
            jax.ShapeDtypeStruct((NTOT, 16), F32),
        ],
    )(xf, aga, agb, pf16, vf16, u1a, u1ba, u1bb, fb1, u2, fb2, u3, fb3,
      z1, zb1, z2, zb2, z3, zb3)


_NC = 2
_NS = 16
_NW = _NC * _NS
_GC = 80
_EPW = E // _NW
_GNCH = _EPW // _GC


_EPT = E // _NS
_GCH = 400
_GSUB = 80


def _gather_body(p_hbm, q_hbm, pf_hbm, src_hbm, dst_hbm,
                 hp_out, hq_out, fd_out, fs_out,
                 idxbuf, hbuf, fbuf, s0, s1):
    cid = lax.axis_index("c")
    sid = lax.axis_index("s")

    def pump(idx_hbm, tab_hbm, out_h, out_f):
        pltpu.sync_copy(idx_hbm.at[pl.ds(sid * _EPT, _EPT)], idxbuf)

        def chunk(j, carry):
            off = j * _GCH
            cps = []
            for k in range(_GCH // _GSUB):
                sl = pl.ds(off + k * _GSUB, _GSUB)
                dsl = pl.ds(k * _GSUB, _GSUB)
                cps.append(pltpu.async_copy(
                    tab_hbm.at[idxbuf.at[sl]], hbuf.at[dsl], s0))
                cps.append(pltpu.async_copy(
                    pf_hbm.at[idxbuf.at[sl]], fbuf.at[dsl], s1))
            for cp in cps:
                cp.wait()
            base = sid * _EPT + off
            pltpu.sync_copy(hbuf, out_h.at[pl.ds(base, _GCH)])
            pltpu.sync_copy(fbuf, out_f.at[pl.ds(base, _GCH)])
            return carry

        lax.fori_loop(0, _EPT // _GCH, chunk, 0)

    @pl.when(cid == 0)
    def _core0():
        pump(dst_hbm, p_hbm, hp_out, fd_out)

    @pl.when(cid == 1)
    def _core1():
        pump(src_hbm, q_hbm, hq_out, fs_out)


def _gather_stage(p2, q2, pf16f, src, dst):
    mesh = plsc.VectorSubcoreMesh(core_axis_name="c", subcore_axis_name="s")
    k = pl.kernel(
        _gather_body, mesh=mesh,
        compiler_params=pltpu.CompilerParams(use_tc_tiling_on_sc=False),
        out_type=[
            jax.ShapeDtypeStruct((E, HID), F32),
            jax.ShapeDtypeStruct((E, HID), F32),
            jax.ShapeDtypeStruct((E, 16), F32),
            jax.ShapeDtypeStruct((E, 16), F32),
        ],
        scratch_types=[
            pltpu.VMEM((_EPT,), jnp.int32),
            pltpu.VMEM((_GCH, HID), F32),
            pltpu.VMEM((_GCH, 16), F32),
            pltpu.SemaphoreType.DMA,
            pltpu.SemaphoreType.DMA,
        ],
    )
    return k(p2, q2, pf16f, src, dst)


_SC_ROWS = 40960
_SC_RPT = _SC_ROWS // _NS
_SCH = 80
_SNCH = _EPT // _SCH
_AGW = 40


def _scatter_body(mpa_hbm, mpb_hbm, dst_hbm, aga_out, agb_out,
                  didx, mpbuf, zbuf, shared, si0, si1, sm0, sm1):
    cid = lax.axis_index("c")
    sid = lax.axis_index("s")
    si = (si0, si1)
    sm = (sm0, sm1)

    def zrow(r, c):
        z = jnp.zeros((16,), F32)
        zbuf[r, pl.ds(0, 16)] = z
        zbuf[r, pl.ds(16, 16)] = z
        zbuf[r, pl.ds(24, 16)] = z
        return c

    lax.fori_loop(0, 320, zrow, 0)
    for k in range(_SC_RPT // 320):
        pltpu.sync_copy(zbuf, shared.at[pl.ds(sid * _SC_RPT + k * 320, 320)])
    plsc.subcore_barrier()

    def pump(mp_hbm):
        tbase = sid * _EPT
        for b in (0, 1):
            pltpu.async_copy(dst_hbm.at[pl.ds(tbase + b * _SCH, _SCH)],
                             didx.at[b], si[b])
            pltpu.async_copy(mp_hbm.at[pl.ds(tbase + b * _SCH, _SCH)],
                             mpbuf.at[b], sm[b])

        def outer(o, carry):
            for b in (0, 1):
                g = 2 * o + b
                pltpu.make_async_copy(dst_hbm.at[pl.ds(0, _SCH)],
                                      didx.at[b], si[b]).wait()
                pltpu.make_async_copy(mp_hbm.at[pl.ds(0, _SCH)],
                                      mpbuf.at[b], sm[b]).wait()
                pltpu.sync_copy(mpbuf.at[b], shared.at[didx.at[b]], add=True)

                @pl.when(g + 2 < _SNCH)
                def _prefetch():
                    nbase = tbase + (g + 2) * _SCH
                    pltpu.async_copy(dst_hbm.at[pl.ds(nbase, _SCH)],
                                     didx.at[b], si[b])
                    pltpu.async_copy(mp_hbm.at[pl.ds(nbase, _SCH)],
                                     mpbuf.at[b], sm[b])
            return carry

        lax.fori_loop(0, _SNCH // 2, outer, 0)

    @pl.when(cid == 0)
    def _core0():
        pump(mpa_hbm)

    @pl.when(cid == 1)
    def _core1():
        pump(mpb_hbm)

    plsc.subcore_barrier()

    for k in range(_SC_RPT // 320):
        row = sid * _SC_RPT + k * 320

        @pl.when(row < NTOT)
        def _cp():
            @pl.when(cid == 0)
            def _a():
                pltpu.sync_copy(shared.at[pl.ds(row, 320)],
                                aga_out.at[pl.ds(row, 320)])

            @pl.when(cid == 1)
            def _b():
                pltpu.sync_copy(shared.at[pl.ds(row, 320)],
                                agb_out.at[pl.ds(row, 320)])


def _scatter_stage(mpa, mpb, dst):
    mesh = plsc.VectorSubcoreMesh(core_axis_name="c", subcore_axis_name="s")
    k = pl.kernel(
        _scatter_body, mesh=mesh,
        compiler_params=pltpu.CompilerParams(use_tc_tiling_on_sc=False),
        out_type=[
            jax.ShapeDtypeStruct((NTOT, _AGW), F32),
            jax.ShapeDtypeStruct((NTOT, _AGW), F32),
        ],
        scratch_types=[
            pltpu.VMEM((2, _SCH), jnp.int32),
            pltpu.VMEM((2, _SCH, _AGW), F32),
            pltpu.VMEM((320, _AGW), F32),
            pltpu.VMEM_SHARED((_SC_ROWS, _AGW), F32),
            pltpu.SemaphoreType.DMA,
            pltpu.SemaphoreType.DMA,
            pltpu.SemaphoreType.DMA,
            pltpu.SemaphoreType.DMA,
        ],
    )
    return k(mpa, mpb, dst)


def kernel(x, pos, vel, edge_index, edge_attr, params):
    w0r = params['weight_scalar_r'][:, :, 0]
    w1r = params['weight_scalar_r'][:, :, 1]
    w1i = params['weight_scalar_i'][:, :, 1]
    wvr8 = params['weight_vector_r'].reshape(8)
    wvi8 = params['weight_vector_i'].reshape(8)

    (mw1, mb1), (mw2, mb2), (mw3, mb3) = params['message_net']
    wd = mw1[0:D]
    wq = mw1[D:2 * D]
    wdist = mw1[2 * D:2 * D + 1]
    wea = mw1[2 * D + 1:]

    (pv1, pc1), (pv2, pc2), (pv3, pc3) = params['update_pos_net']
    (fu1, fb1), (fu2, fb2), (fu3, fb3) = params['update_feat_net']
    u1a = fu1[0:D]
    u1b = fu1[D:]
    (zv1, zb1), (zv2, zb2), (zv3, zb3) = params['update_vel_net']

    row = lambda b: b.reshape(1, -1)

    x2, p, q = _spectral_x(x, w0r, w1r, w1i, wd, wq)
    center = _center(pos)
    pf16, vf16 = _spectral_v(pos, vel, center, wvr8, wvi8)

    xf = x2.reshape(NTOT, D)
    p2 = p.reshape(NTOT, HID)
    q2 = q.reshape(NTOT, HID)
    pf16f = pf16.reshape(NTOT, 16)
    vf16f = vf16.reshape(NTOT, 16)
    src = edge_index[0]
    dst = edge_index[1]
    ea = edge_attr.reshape(E, D_EDGE)

    hp, hq, dpd, dps = _gather_stage(p2, q2, pf16f, src, dst)

    mpa, mpb = _edge_mlp(hp, hq, dpd, dps, ea, row(wdist.reshape(-1)), wea,
                         row(mb1), mw2, row(mb2), mw3, row(mb3),
                         pv1, row(pc1), pv2, row(pc2), pv3, row(pc3))

    aga, agb = _scatter_stage(mpa, mpb, dst)

    xn, pn16, vn16 = _node_update(
        xf, aga, agb, pf16f, vf16f,
        u1a, u1b[0:40], u1b[40:64], row(fb1), fu2, row(fb2), fu3, row(fb3),
        zv1, row(zb1), zv2, row(zb2), zv3, row(zb3))

    x_new = xn.reshape(T, N, D)
    pos_new = pn16[:, 0:POS].reshape(T, N, POS)
    vel_new = vn16[:, 0:POS].reshape(T, N, POS)
    return (x_new, pos_new, vel_new)


# speedup vs baseline: 1.1268x; 1.0025x over previous
import functools

import jax
import jax.numpy as jnp
from jax import lax
from jax.experimental import pallas as pl
from jax.experimental.pallas import tpu as pltpu
from jax.experimental.pallas import tpu_sc as plsc

T, N, D = 4, 10000, 128
E = 320000
D_EDGE = 16
POS = 3
HID = 64
NTOT = T * N

F32 = jnp.float32


def _silu(v):
    return v / (1.0 + jnp.exp(-v))


def _dot(a, b):
    return jnp.dot(a, b, preferred_element_type=F32)


def _spectral_x_body(x_ref, w0r_ref, w1r_ref, w1i_ref, wd_ref, wq_ref,
                     x2_ref, p_ref, q_ref):
    x0 = x_ref[0]
    x1 = x_ref[1]
    x2 = x_ref[2]
    x3 = x_ref[3]
    f0 = x0 + x1 + x2 + x3
    a = x0 - x2
    b = x3 - x1
    r0 = _dot(f0, w0r_ref[...])
    r1 = _dot(a, w1r_ref[...]) - _dot(b, w1i_ref[...])
    i1 = _dot(a, w1i_ref[...]) + _dot(b, w1r_ref[...])
    y0 = 0.25 * (r0 + 2.0 * r1)
    y1 = 0.25 * (r0 - 2.0 * i1)
    y2 = 0.25 * (r0 - 2.0 * r1)
    y3 = 0.25 * (r0 + 2.0 * i1)
    o0 = x0 + y0
    o1 = x1 + y1
    o2 = x2 + y2
    o3 = x3 + y3
    x2_ref[0] = o0
    x2_ref[1] = o1
    x2_ref[2] = o2
    x2_ref[3] = o3
    wd = wd_ref[...]
    wq = wq_ref[...]
    p_ref[0] = _dot(o0, wd)
    p_ref[1] = _dot(o1, wd)
    p_ref[2] = _dot(o2, wd)
    p_ref[3] = _dot(o3, wd)
    q_ref[0] = _dot(o0, wq)
    q_ref[1] = _dot(o1, wq)
    q_ref[2] = _dot(o2, wq)
    q_ref[3] = _dot(o3, wq)


def _spectral_x(x, w0r, w1r, w1i, wd, wq):
    nb = 1000
    grid = N // nb
    full = lambda shape: pl.BlockSpec(shape, lambda i: (0,) * len(shape))
    return pl.pallas_call(
        _spectral_x_body,
        grid=(grid,),
        in_specs=[
            pl.BlockSpec((T, nb, D), lambda i: (0, i, 0)),
            full((D, D)), full((D, D)), full((D, D)),
            full((D, HID)), full((D, HID)),
        ],
        out_specs=[
            pl.BlockSpec((T, nb, D), lambda i: (0, i, 0)),
            pl.BlockSpec((T, nb, HID), lambda i: (0, i, 0)),
            pl.BlockSpec((T, nb, HID), lambda i: (0, i, 0)),
        ],
        out_shape=[
            jax.ShapeDtypeStruct((T, N, D), F32),
            jax.ShapeDtypeStruct((T, N, HID), F32),
            jax.ShapeDtypeStruct((T, N, HID), F32),
        ],
    )(x, w0r, w1r, w1i, wd, wq)


def _center_body(pos_ref, out_ref):
    i = pl.program_id(0)

    @pl.when(i == 0)
    def _init():
        out_ref[...] = jnp.zeros_like(out_ref)

    part = jnp.sum(pos_ref[...], axis=1)
    out_ref[...] += part * (1.0 / N)


def _center(pos):
    nb = 1000
    return pl.pallas_call(
        _center_body,
        grid=(N // nb,),
        in_specs=[pl.BlockSpec((T, nb, POS), lambda i: (0, i, 0))],
        out_specs=pl.BlockSpec((T, POS), lambda i: (0, 0)),
        out_shape=jax.ShapeDtypeStruct((T, POS), F32),
    )(pos)


def _spectral_v_body(pos_ref, vel_ref, c_ref, wr_ref, wi_ref, pf_ref, vf_ref):
    center = []
    pc = []
    vv = []
    for t in range(T):
        ct = c_ref[t]
        center.append(ct)
        pc.append(pos_ref[t] - ct)
        vv.append(vel_ref[t])
    f0p = pc[0] + pc[1] + pc[2] + pc[3]
    f0v = vv[0] + vv[1] + vv[2] + vv[3]
    ap = pc[0] - pc[2]
    av = vv[0] - vv[2]
    bp = pc[3] - pc[1]
    bv = vv[3] - vv[1]

    def mix(arr_p, arr_v, w_ref, o, m):
        return arr_p * w_ref[4 * 0 + 2 * o + m] + arr_v * w_ref[4 * 1 + 2 * o + m]

    for o in range(2):
        r0 = mix(f0p, f0v, wr_ref, o, 0)
        r1 = mix(ap, av, wr_ref, o, 1) - mix(bp, bv, wi_ref, o, 1)
        i1 = mix(ap, av, wi_ref, o, 1) + mix(bp, bv, wr_ref, o, 1)
        y = (0.25 * (r0 + 2.0 * r1), 0.25 * (r0 - 2.0 * i1),
             0.25 * (r0 - 2.0 * r1), 0.25 * (r0 + 2.0 * i1))
        for t in range(T):
            if o == 0:
                pf_ref[t, :, 0:POS] = pc[t] + y[t] + center[t]
            else:
                vf_ref[t, :, 0:POS] = vv[t] + y[t]
    nb = pf_ref.shape[1]
    pf_ref[:, :, POS:] = jnp.zeros((T, nb, 16 - POS), F32)
    vf_ref[:, :, POS:] = jnp.zeros((T, nb, 16 - POS), F32)


def _spectral_v(pos, vel, center, wr8, wi8):
    nb = 1000
    return pl.pallas_call(
        _spectral_v_body,
        grid=(N // nb,),
        in_specs=[
            pl.BlockSpec((T, nb, POS), lambda i: (0, i, 0)),
            pl.BlockSpec((T, nb, POS), lambda i: (0, i, 0)),
            pl.BlockSpec((T, POS), lambda i: (0, 0)),
            pl.BlockSpec(memory_space=pltpu.SMEM),
            pl.BlockSpec(memory_space=pltpu.SMEM),
        ],
        out_specs=[
            pl.BlockSpec((T, nb, 16), lambda i: (0, i, 0)),
            pl.BlockSpec((T, nb, 16), lambda i: (0, i, 0)),
        ],
        out_shape=[
            jax.ShapeDtypeStruct((T, N, 16), F32),
            jax.ShapeDtypeStruct((T, N, 16), F32),
        ],
    )(pos, vel, center, wr8, wi8)


def _edge_body(hp_ref, hq_ref, dpd_ref, dps_ref, ea_ref, wdist_ref, wea_ref,
               b1_ref, w2_ref, b2_ref, w3_ref, b3_ref,
               v1_ref, c1_ref, v2_ref, c2_ref, v3_ref, c3_ref,
               mpa_ref, mpb_ref):
    diff = dpd_ref[...] - dps_ref[...]
    dist = jnp.sqrt(jnp.sum(diff * diff, axis=1, keepdims=True) + 1e-12)
    u = hp_ref[...] + hq_ref[...] + _dot(ea_ref[...], wea_ref[...]) \
        + dist * wdist_ref[...] + b1_ref[...]
    u = _silu(u)
    u = _silu(_dot(u, w2_ref[...]) + b2_ref[...])
    m = _dot(u, w3_ref[...]) + b3_ref[...]
    p = _silu(_dot(m, v1_ref[...]) + c1_ref[...])
    p = _silu(_dot(p, v2_ref[...]) + c2_ref[...])
    s = _dot(p, v3_ref[...]) + c3_ref[...]
    mpa_ref[...] = m[:, 0:40]
    mpb_ref[:, 0:24] = m[:, 40:64]
    mpb_ref[:, 24:40] = diff * s


def _edge_mlp(hp, hq, dpd, dps, ea, wdist, wea, b1, w2, b2, w3, b3,
              v1, c1, v2, c2, v3, c3):
    eb = 5000
    grid = E // eb
    full = lambda shape: pl.BlockSpec(shape, lambda i: (0,) * len(shape))
    return pl.pallas_call(
        _edge_body,
        grid=(grid,),
        in_specs=[
            pl.BlockSpec((eb, HID), lambda i: (i, 0)),
            pl.BlockSpec((eb, HID), lambda i: (i, 0)),
            pl.BlockSpec((eb, 16), lambda i: (i, 0)),
            pl.BlockSpec((eb, 16), lambda i: (i, 0)),
            pl.BlockSpec((eb, D_EDGE), lambda i: (i, 0)),
            full((1, HID)), full((D_EDGE, HID)), full((1, HID)),
            full((HID, HID)), full((1, HID)),
            full((HID, HID)), full((1, HID)),
            full((HID, HID)), full((1, HID)),
            full((HID, HID)), full((1, HID)),
            full((HID, 1)), full((1, 1)),
        ],
        out_specs=[
            pl.BlockSpec((eb, _AGW), lambda i: (i, 0)),
            pl.BlockSpec((eb, _AGW), lambda i: (i, 0)),
        ],
        out_shape=[
            jax.ShapeDtypeStruct((E, _AGW), F32),
            jax.ShapeDtypeStruct((E, _AGW), F32),
        ],
    )(hp, hq, dpd, dps, ea, wdist, wea, b1, w2, b2, w3, b3,
      v1, c1, v2, c2, v3, c3)


def _node_body(xf_ref, aga_ref, agb_ref, pf_ref, vf_ref,
               u1a_ref, u1ba_ref, u1bb_ref, fb1_ref, u2_ref, fb2_ref,
               u3_ref, fb3_ref,
               z1_ref, zb1_ref, z2_ref, zb2_ref, z3_ref, zb3_ref,
               xn_ref, pn_ref, vn_ref):
    xf = xf_ref[...]
    aga = aga_ref[...]
    agb = agb_ref[...]
    ap = agb[:, 24:40]
    h = _silu(_dot(xf, u1a_ref[...]) + _dot(aga, u1ba_ref[...])
              + _dot(agb[:, 0:24], u1bb_ref[...]) + fb1_ref[...])
    h = _silu(_dot(h, u2_ref[...]) + fb2_ref[...])
    xn_ref[...] = _dot(h, u3_ref[...]) + fb3_ref[...]
    z = _silu(_dot(xf, z1_ref[...]) + zb1_ref[...])
    z = _silu(_dot(z, z2_ref[...]) + zb2_ref[...])
    s = _dot(z, z3_ref[...]) + zb3_ref[...]
    vn = s * vf_ref[...] + ap
    vn_ref[...] = vn
    pn_ref[...] = pf_ref[...] + vn


def _node_update(xf, aga, agb, pf16, vf16, u1a, u1ba, u1bb, fb1, u2, fb2,
                 u3, fb3, z1, zb1, z2, zb2, z3, zb3):
    nb = 2000
    grid = NTOT // nb
    full = lambda shape: pl.BlockSpec(shape, lambda i: (0,) * len(shape))
    return pl.pallas_call(
        _node_body,
        grid=(grid,),
        in_specs=[
            pl.BlockSpec((nb, D), lambda i: (i, 0)),
            pl.BlockSpec((nb, _AGW), lambda i: (i, 0)),
            pl.BlockSpec((nb, _AGW), lambda i: (i, 0)),
            pl.BlockSpec((nb, 16), lambda i: (i, 0)),
            pl.BlockSpec((nb, 16), lambda i: (i, 0)),
            full((D, HID)), full((_AGW, HID)), full((24, HID)),
            full((1, HID)),
            full((HID, HID)), full((1, HID)),
            full((HID, D)), full((1, D)),
            full((D, HID)), full((1, HID)),
            full((HID, HID)), full((1, HID)),
            full((HID, 1)), full((1, 1)),
        ],
        out_specs=[
            pl.BlockSpec((nb, D), lambda i: (i, 0)),
            pl.BlockSpec((nb, 16), lambda i: (i, 0)),
            pl.BlockSpec((nb, 16), lambda i: (i, 0)),
        ],
        out_shape=[
            jax.ShapeDtypeStruct((NTOT, D), F32),
            jax.ShapeDtypeStruct((NTOT, 16), F32),
            jax.ShapeDtypeStruct((NTOT, 16), F32),
        ],
    )(xf, aga, agb, pf16, vf16, u1a, u1ba, u1bb, fb1, u2, fb2, u3, fb3,
      z1, zb1, z2, zb2, z3, zb3)


_NC = 2
_NS = 16
_NW = _NC * _NS
_GC = 80
_EPW = E // _NW
_GNCH = _EPW // _GC


_EPT = E // _NS
_GCH = 400
_GSUB = 80


def _gather_body(p_hbm, q_hbm, pf_hbm, src_hbm, dst_hbm,
                 hp_out, hq_out, fd_out, fs_out,
                 idxbuf, hbuf, fbuf, s0, s1):
    cid = lax.axis_index("c")
    sid = lax.axis_index("s")

    def pump(idx_hbm, tab_hbm, out_h, out_f):
        pltpu.sync_copy(idx_hbm.at[pl.ds(sid * _EPT, _EPT)], idxbuf)

        def chunk(j, carry):
            off = j * _GCH
            cps = []
            for k in range(_GCH // _GSUB):
                sl = pl.ds(off + k * _GSUB, _GSUB)
                dsl = pl.ds(k * _GSUB, _GSUB)
                cps.append(pltpu.async_copy(
                    tab_hbm.at[idxbuf.at[sl]], hbuf.at[dsl], s0))
                cps.append(pltpu.async_copy(
                    pf_hbm.at[idxbuf.at[sl]], fbuf.at[dsl], s1))
            for cp in cps:
                cp.wait()
            base = sid * _EPT + off
            pltpu.sync_copy(hbuf, out_h.at[pl.ds(base, _GCH)])
            pltpu.sync_copy(fbuf, out_f.at[pl.ds(base, _GCH)])
            return carry

        lax.fori_loop(0, _EPT // _GCH, chunk, 0)

    @pl.when(cid == 0)
    def _core0():
        pump(dst_hbm, p_hbm, hp_out, fd_out)

    @pl.when(cid == 1)
    def _core1():
        pump(src_hbm, q_hbm, hq_out, fs_out)


def _gather_stage(p2, q2, pf16f, src, dst):
    mesh = plsc.VectorSubcoreMesh(core_axis_name="c", subcore_axis_name="s")
    k = pl.kernel(
        _gather_body, mesh=mesh,
        compiler_params=pltpu.CompilerParams(use_tc_tiling_on_sc=False),
        out_type=[
            jax.ShapeDtypeStruct((E, HID), F32),
            jax.ShapeDtypeStruct((E, HID), F32),
            jax.ShapeDtypeStruct((E, 16), F32),
            jax.ShapeDtypeStruct((E, 16), F32),
        ],
        scratch_types=[
            pltpu.VMEM((_EPT,), jnp.int32),
            pltpu.VMEM((_GCH, HID), F32),
            pltpu.VMEM((_GCH, 16), F32),
            pltpu.SemaphoreType.DMA,
            pltpu.SemaphoreType.DMA,
        ],
    )
    return k(p2, q2, pf16f, src, dst)


_SC_ROWS = 40960
_SC_RPT = _SC_ROWS // _NS
_SCH = 80
_SNCH = _EPT // _SCH
_AGW = 40


def _scatter_body(mpa_hbm, mpb_hbm, dst_hbm, aga_out, agb_out,
                  didx, mpbuf, zbuf, shared, si0, si1, sm0, sm1):
    cid = lax.axis_index("c")
    sid = lax.axis_index("s")
    si = (si0, si1)
    sm = (sm0, sm1)

    def zrow(r, c):
        z = jnp.zeros((16,), F32)
        zbuf[r, pl.ds(0, 16)] = z
        zbuf[r, pl.ds(16, 16)] = z
        zbuf[r, pl.ds(24, 16)] = z
        return c

    lax.fori_loop(0, 320, zrow, 0)
    for k in range(_SC_RPT // 320):
        pltpu.sync_copy(zbuf, shared.at[pl.ds(sid * _SC_RPT + k * 320, 320)])
    plsc.subcore_barrier()

    def pump(mp_hbm):
        tbase = sid * _EPT
        for b in (0, 1):
            pltpu.async_copy(dst_hbm.at[pl.ds(tbase + b * _SCH, _SCH)],
                             didx.at[b], si[b])
            pltpu.async_copy(mp_hbm.at[pl.ds(tbase + b * _SCH, _SCH)],
                             mpbuf.at[b], sm[b])

        def outer(o, carry):
            for b in (0, 1):
                g = 2 * o + b
                pltpu.make_async_copy(dst_hbm.at[pl.ds(0, _SCH)],
                                      didx.at[b], si[b]).wait()
                pltpu.make_async_copy(mp_hbm.at[pl.ds(0, _SCH)],
                                      mpbuf.at[b], sm[b]).wait()
                pltpu.sync_copy(mpbuf.at[b], shared.at[didx.at[b]], add=True)

                @pl.when(g + 2 < _SNCH)
                def _prefetch():
                    nbase = tbase + (g + 2) * _SCH
                    pltpu.async_copy(dst_hbm.at[pl.ds(nbase, _SCH)],
                                     didx.at[b], si[b])
                    pltpu.async_copy(mp_hbm.at[pl.ds(nbase, _SCH)],
                                     mpbuf.at[b], sm[b])
            return carry

        lax.fori_loop(0, _SNCH // 2, outer, 0)

    @pl.when(cid == 0)
    def _core0():
        pump(mpa_hbm)

    @pl.when(cid == 1)
    def _core1():
        pump(mpb_hbm)

    plsc.subcore_barrier()

    for k in range(_SC_RPT // 320):
        row = sid * _SC_RPT + k * 320

        @pl.when(row < NTOT)
        def _cp():
            @pl.when(cid == 0)
            def _a():
                pltpu.sync_copy(shared.at[pl.ds(row, 320)],
                                aga_out.at[pl.ds(row, 320)])

            @pl.when(cid == 1)
            def _b():
                pltpu.sync_copy(shared.at[pl.ds(row, 320)],
                                agb_out.at[pl.ds(row, 320)])


def _scatter_stage(mpa, mpb, dst):
    mesh = plsc.VectorSubcoreMesh(core_axis_name="c", subcore_axis_name="s")
    k = pl.kernel(
        _scatter_body, mesh=mesh,
        compiler_params=pltpu.CompilerParams(use_tc_tiling_on_sc=False),
        out_type=[
            jax.ShapeDtypeStruct((NTOT, _AGW), F32),
            jax.ShapeDtypeStruct((NTOT, _AGW), F32),
        ],
        scratch_types=[
            pltpu.VMEM((2, _SCH), jnp.int32),
            pltpu.VMEM((2, _SCH, _AGW), F32),
            pltpu.VMEM((320, _AGW), F32),
            pltpu.VMEM_SHARED((_SC_ROWS, _AGW), F32),
            pltpu.SemaphoreType.DMA,
            pltpu.SemaphoreType.DMA,
            pltpu.SemaphoreType.DMA,
            pltpu.SemaphoreType.DMA,
        ],
    )
    return k(mpa, mpb, dst)


def kernel(x, pos, vel, edge_index, edge_attr, params):
    w0r = params['weight_scalar_r'][:, :, 0]
    w1r = params['weight_scalar_r'][:, :, 1]
    w1i = params['weight_scalar_i'][:, :, 1]
    wvr8 = params['weight_vector_r'].reshape(8)
    wvi8 = params['weight_vector_i'].reshape(8)

    (mw1, mb1), (mw2, mb2), (mw3, mb3) = params['message_net']
    wd = mw1[0:D]
    wq = mw1[D:2 * D]
    wdist = mw1[2 * D:2 * D + 1]
    wea = mw1[2 * D + 1:]

    (pv1, pc1), (pv2, pc2), (pv3, pc3) = params['update_pos_net']
    (fu1, fb1), (fu2, fb2), (fu3, fb3) = params['update_feat_net']
    u1a = fu1[0:D]
    u1b = fu1[D:]
    (zv1, zb1), (zv2, zb2), (zv3, zb3) = params['update_vel_net']

    row = lambda b: b.reshape(1, -1)

    x2, p, q = _spectral_x(x, w0r, w1r, w1i, wd, wq)
    center = _center(pos)
    pf16, vf16 = _spectral_v(pos, vel, center, wvr8, wvi8)

    xf = x2.reshape(NTOT, D)
    p2 = p.reshape(NTOT, HID)
    q2 = q.reshape(NTOT, HID)
    pf16f = pf16.reshape(NTOT, 16)
    vf16f = vf16.reshape(NTOT, 16)
    src = edge_index[0]
    dst = edge_index[1]
    ea = edge_attr.reshape(E, D_EDGE)

    hp, hq, dpd, dps = _gather_stage(p2, q2, pf16f, src, dst)

    mpa, mpb = _edge_mlp(hp, hq, dpd, dps, ea, row(wdist.reshape(-1)), wea,
                         row(mb1), mw2, row(mb2), mw3, row(mb3),
                         pv1, row(pc1), pv2, row(pc2), pv3, row(pc3))

    aga, agb = _scatter_stage(mpa, mpb, dst)

    xn, pn16, vn16 = _node_update(
        xf, aga, agb, pf16f, vf16f,
        u1a, u1b[0:40], u1b[40:64], row(fb1), fu2, row(fb2), fu3, row(fb3),
        zv1, row(zb1), zv2, row(zb2), zv3, row(zb3))

    x_new = xn.reshape(T, N, D)
    pos_new = pn16[:, 0:POS].reshape(T, N, POS)
    vel_new = vn16[:, 0:POS].reshape(T, N, POS)
    return (x_new, pos_new, vel_new)
